# baseline (speedup 1.0000x reference)
import functools

import jax
import jax.numpy as jnp
from jax import lax
from jax.experimental import pallas as pl
from jax.experimental.pallas import tpu as pltpu
from jax.experimental.pallas import tpu_sc as plsc

VOCAB = 1000000
EMB = 64
HID = 256
B = 4096
L = 200

NC = 2
NS = 16
NW = NC * NS
ROWS_PER_W = B // NW
CH = 104
L_PAD = 2 * CH


def _sc_gather_sum(idx3, table):
    mesh = plsc.VectorSubcoreMesh(core_axis_name="c", subcore_axis_name="s")

    NBUF = 4

    @functools.partial(
        pl.kernel,
        mesh=mesh,
        out_type=jax.ShapeDtypeStruct((B, EMB), jnp.float32),
        compiler_params=pltpu.CompilerParams(use_tc_tiling_on_sc=False),
        scratch_types=[
            pltpu.VMEM((ROWS_PER_W, 2, CH), jnp.int32),
            pltpu.VMEM((NBUF, L_PAD, EMB), jnp.float32),
            pltpu.VMEM((ROWS_PER_W, EMB), jnp.float32),
            [pltpu.SemaphoreType.DMA] * NBUF,
        ],
    )
    def k(idx_hbm, table_hbm, out_hbm, idx_v, rows_v, out_v, sems):
        wid = lax.axis_index("s") * NC + lax.axis_index("c")
        base = wid * ROWS_PER_W

        pltpu.sync_copy(idx_hbm.at[pl.ds(base, ROWS_PER_W)], idx_v)

        def issue(row, b):
            pltpu.async_copy(
                table_hbm.at[idx_v.at[row, 0]],
                rows_v.at[b, pl.ds(0, CH)],
                sems[b],
            )
            pltpu.async_copy(
                table_hbm.at[idx_v.at[row, 1]],
                rows_v.at[b, pl.ds(CH, CH)],
                sems[b],
            )

        def wait(b):
            pltpu.make_async_copy(
                table_hbm.at[pl.ds(0, L_PAD)], rows_v.at[b], sems[b]
            ).wait()

        def accum(row, b):
            accs = [jnp.zeros((16,), jnp.float32) for _ in range(8)]
            for j in range(L):
                p = (j % 2) * 4
                for g in range(4):
                    accs[p + g] = accs[p + g] + rows_v[b, j, pl.ds(16 * g, 16)]
            for g in range(4):
                out_v[row, pl.ds(16 * g, 16)] = accs[g] + accs[g + 4]

        for b in range(NBUF):
            issue(b, b)

        def outer(i, carry):
            for b in range(NBUF):
                row = NBUF * i + b
                wait(b)
                nxt = row + NBUF

                @pl.when(nxt < ROWS_PER_W)
                def _():
                    issue(nxt, b)

                accum(row, b)
            return carry

        lax.fori_loop(0, ROWS_PER_W // NBUF, outer, 0)

        pltpu.sync_copy(out_v, out_hbm.at[pl.ds(base, ROWS_PER_W)])

    return k(idx3, table)


def _mlp_body(x_ref, w1_ref, b1_ref, w2_ref, b2_ref, o_ref):
    x = x_ref[...] * jnp.float32(1.0 / L)
    h = jnp.dot(x, w1_ref[...], preferred_element_type=jnp.float32,
                precision=lax.Precision.HIGHEST)
    h = jnp.maximum(h + b1_ref[...], 0.0)
    z = jnp.dot(h, w2_ref[...], preferred_element_type=jnp.float32,
                precision=lax.Precision.HIGHEST)
    z = z + b2_ref[...]
    m = jnp.max(z, axis=1, keepdims=True)
    s = z - m
    lse = jnp.log(jnp.sum(jnp.exp(s), axis=1, keepdims=True))
    o_ref[...] = s - lse


def _mlp(sums, W1, b1, W2, b2):
    W2p = jnp.pad(W2, ((0, 0), (0, 128 - W2.shape[1])))
    b2p = jnp.pad(b2, (0, 128 - b2.shape[0]), constant_values=-1e30)
    grid = 4
    blk = B // grid
    out = pl.pallas_call(
        _mlp_body,
        grid=(grid,),
        in_specs=[
            pl.BlockSpec((blk, EMB), lambda i: (i, 0)),
            pl.BlockSpec((EMB, HID), lambda i: (0, 0)),
            pl.BlockSpec((1, HID), lambda i: (0, 0)),
            pl.BlockSpec((HID, 128), lambda i: (0, 0)),
            pl.BlockSpec((1, 128), lambda i: (0, 0)),
        ],
        out_specs=pl.BlockSpec((blk, 128), lambda i: (i, 0)),
        out_shape=jax.ShapeDtypeStruct((B, 128), jnp.float32),
    )(sums, W1, b1.reshape(1, HID), W2p, b2p.reshape(1, 128))
    return out[:, :2]


def kernel(input_idxs, table, W1, b1, W2, b2):
    idx3 = jnp.pad(input_idxs.astype(jnp.int32), ((0, 0), (0, L_PAD - L)))
    idx3 = idx3.reshape(B, 2, CH)
    sums = _sc_gather_sum(idx3, table)
    return _mlp(sums, W1, b1, W2, b2)


# speedup vs baseline: 1.2887x; 1.2887x over previous
import functools

import jax
import jax.numpy as jnp
from jax import lax
from jax.experimental import pallas as pl
from jax.experimental.pallas import tpu as pltpu
from jax.experimental.pallas import tpu_sc as plsc

VOCAB = 1000000
EMB = 64
HID = 256
B = 4096
L = 200

NC = 2
NS = 16
NW = NC * NS
ROWS_PER_W = B // NW
CH = 104
L_PAD = 2 * CH


def _sc_gather_sum(idx3, table):
    mesh = plsc.VectorSubcoreMesh(core_axis_name="c", subcore_axis_name="s")

    NBUF = 2
    PAIRS_PER_W = ROWS_PER_W // 2

    @functools.partial(
        pl.kernel,
        mesh=mesh,
        out_type=jax.ShapeDtypeStruct((B, EMB), jnp.float32),
        compiler_params=pltpu.CompilerParams(use_tc_tiling_on_sc=False),
        scratch_types=[
            pltpu.VMEM((PAIRS_PER_W, 2 * L_PAD), jnp.int32),
            pltpu.VMEM((NBUF, 2 * L_PAD, EMB), jnp.float32),
            pltpu.VMEM((ROWS_PER_W, EMB), jnp.float32),
            [pltpu.SemaphoreType.DMA] * NBUF,
        ],
    )
    def k(idx_hbm, table_hbm, out_hbm, idx_v, rows_v, out_v, sems):
        wid = lax.axis_index("s") * NC + lax.axis_index("c")
        base = wid * PAIRS_PER_W

        pltpu.sync_copy(idx_hbm.at[pl.ds(base, PAIRS_PER_W)], idx_v)

        def issue(pair, b):
            pltpu.async_copy(
                table_hbm.at[idx_v.at[pair]],
                rows_v.at[b],
                sems[b],
            )

        def wait(b):
            pltpu.make_async_copy(
                table_hbm.at[pl.ds(0, 2 * L_PAD)], rows_v.at[b], sems[b]
            ).wait()

        def accum(pair, b):
            for half in range(2):
                off = half * L_PAD
                accs = [jnp.zeros((16,), jnp.float32) for _ in range(8)]
                for j in range(L):
                    p = (j % 2) * 4
                    for g in range(4):
                        accs[p + g] = accs[p + g] + rows_v[
                            b, off + j, pl.ds(16 * g, 16)
                        ]
                row = 2 * pair + half
                for g in range(4):
                    out_v[row, pl.ds(16 * g, 16)] = accs[g] + accs[g + 4]

        for b in range(NBUF):
            issue(b, b)

        def outer(i, carry):
            for b in range(NBUF):
                pair = NBUF * i + b
                wait(b)
                nxt = pair + NBUF

                @pl.when(nxt < PAIRS_PER_W)
                def _():
                    issue(nxt, b)

                accum(pair, b)
            return carry

        lax.fori_loop(0, PAIRS_PER_W // NBUF, outer, 0)

        pltpu.sync_copy(out_v, out_hbm.at[pl.ds(base, PAIRS_PER_W * 2)])

    return k(idx3, table)


def _mlp_body(x_ref, w1_ref, b1_ref, w2_ref, b2_ref, o_ref):
    x = x_ref[...] * jnp.float32(1.0 / L)
    h = jnp.dot(x, w1_ref[...], preferred_element_type=jnp.float32,
                precision=lax.Precision.HIGHEST)
    h = jnp.maximum(h + b1_ref[...], 0.0)
    z = jnp.dot(h, w2_ref[...], preferred_element_type=jnp.float32,
                precision=lax.Precision.HIGHEST)
    z = z + b2_ref[...]
    m = jnp.max(z, axis=1, keepdims=True)
    s = z - m
    lse = jnp.log(jnp.sum(jnp.exp(s), axis=1, keepdims=True))
    o_ref[...] = s - lse


def _mlp(sums, W1, b1, W2, b2):
    W2p = jnp.pad(W2, ((0, 0), (0, 128 - W2.shape[1])))
    b2p = jnp.pad(b2, (0, 128 - b2.shape[0]), constant_values=-1e30)
    grid = 4
    blk = B // grid
    out = pl.pallas_call(
        _mlp_body,
        grid=(grid,),
        in_specs=[
            pl.BlockSpec((blk, EMB), lambda i: (i, 0)),
            pl.BlockSpec((EMB, HID), lambda i: (0, 0)),
            pl.BlockSpec((1, HID), lambda i: (0, 0)),
            pl.BlockSpec((HID, 128), lambda i: (0, 0)),
            pl.BlockSpec((1, 128), lambda i: (0, 0)),
        ],
        out_specs=pl.BlockSpec((blk, 128), lambda i: (i, 0)),
        out_shape=jax.ShapeDtypeStruct((B, 128), jnp.float32),
    )(sums, W1, b1.reshape(1, HID), W2p, b2p.reshape(1, 128))
    return out[:, :2]


def kernel(input_idxs, table, W1, b1, W2, b2):
    idx3 = jnp.pad(input_idxs.astype(jnp.int32), ((0, 0), (0, L_PAD - L)))
    idx3 = idx3.reshape(B // 2, 2 * L_PAD)
    sums = _sc_gather_sum(idx3, table)
    return _mlp(sums, W1, b1, W2, b2)
